# baseline (speedup 1.0000x reference)
import functools

import jax
import jax.numpy as jnp
from jax import lax
from jax.experimental import pallas as pl
from jax.experimental.pallas import tpu as pltpu
from jax.experimental.pallas import tpu_sc as plsc

N = 10000
E = 320000
C = 128
NEG_SLOPE = 0.2

NC = 2
NS = 16
NW = NC * NS
EPW = E // NW
B = 80
NB = EPW // B
RPT = N // NS
DW = 8

_HIGH = jax.lax.Precision.HIGHEST


def _k1_body(x_ref, w_ref, b_ref, ap_ref, at_ref, feat_ref, ep_ref, ept_ref):
    i = pl.program_id(0)
    xb = x_ref[...]
    feat = lax.dot_general(xb, w_ref[...], (((1,), (1,)), ((), ()))) + b_ref[...]
    feat_ref[...] = feat
    ep_ref[...] = lax.dot_general(feat, ap_ref[...], (((1,), (0,)), ((), ())))
    ept = lax.dot_general(at_ref[...], feat, (((1,), (1,)), ((), ())))
    ept_ref[0] = ept


def _k3_body(p_ref, dq_ref, feat_ref, ep_ref, out_ref):
    z = ep_ref[:, 0:1] + ep_ref[:, 1:2]
    es = jnp.exp(jnp.where(z >= 0, z, NEG_SLOPE * z))
    den = dq_ref[0, :, 0:1] + dq_ref[1, :, 0:1] + es
    num = p_ref[0] + p_ref[1] + es * feat_ref[...]
    out_ref[...] = num / den


def _sc_body(ei_hbm, ept_hbm, feat_hbm,
             p_hbm, dq_hbm,
             ep0_v, ep1_v, tarb0, tarb1, srcb0, srcb1, rows0, rows1,
             dsrc0, dsrc1, sidx0, sidx1,
             acc, dacc, semg0, semg1, semi0, semi1,
             sema0, sema1, semd0, semd1):
    cid = lax.axis_index("c")
    sid = lax.axis_index("s")
    wid = sid * NC + cid

    zeros16f = jnp.zeros((16,), jnp.float32)
    zeros16i = jnp.zeros((16,), jnp.int32)
    iota16 = lax.iota(jnp.int32, 16)

    tarb = (tarb0, tarb1)
    srcb = (srcb0, srcb1)
    rows = (rows0, rows1)
    dsrc = (dsrc0, dsrc1)
    sidx = (sidx0, sidx1)
    semg = (semg0, semg1)
    semi = (semi0, semi1)
    sema = (sema0, sema1)
    semd = (semd0, semd1)

    def _zr(i, _):
        for k in range(C // 16):
            rows0[i, pl.ds(k * 16, 16)] = zeros16f
        return 0
    lax.fori_loop(0, B, _zr, 0)

    def _zs(m, _):
        rowz = 2 * m + jnp.right_shift(iota16, 3)
        colz = jnp.bitwise_and(iota16, 7)
        plsc.store_scatter(dsrc0, [rowz, colz], zeros16f)
        plsc.store_scatter(dsrc1, [rowz, colz], zeros16f)
        return 0
    lax.fori_loop(0, B * DW // 16, _zs, 0)

    base = sid * RPT
    for i in range(RPT // B):
        pltpu.sync_copy(rows0, acc.at[pl.ds(base + i * B, B)])
        pltpu.sync_copy(dsrc0, dacc.at[pl.ds(base + i * B, B)])
    rem = RPT % B
    if rem:
        off = base + (RPT // B) * B
        pltpu.sync_copy(rows0.at[pl.ds(0, rem)], acc.at[pl.ds(off, rem)])
        pltpu.sync_copy(dsrc0.at[pl.ds(0, rem)], dacc.at[pl.ds(off, rem)])

    NCH = N // 1000
    for i2 in range(NCH):
        pltpu.async_copy(ept_hbm.at[i2, 0], ep0_v.at[pl.ds(i2 * 1000, 1000)],
                         semg0)
        pltpu.async_copy(ept_hbm.at[i2, 1], ep1_v.at[pl.ds(i2 * 1000, 1000)],
                         semg0)
    for i2 in range(NCH):
        pltpu.make_async_copy(ept_hbm.at[i2, 0],
                              ep0_v.at[pl.ds(i2 * 1000, 1000)], semg0).wait()
        pltpu.make_async_copy(ept_hbm.at[i2, 1],
                              ep1_v.at[pl.ds(i2 * 1000, 1000)], semg0).wait()

    def issue_idx(j, slot):
        pltpu.async_copy(ei_hbm.at[0, wid, j], tarb[slot], semi[slot])
        pltpu.async_copy(ei_hbm.at[1, wid, j], srcb[slot], semi[slot])

    def wait_idx(slot):
        pltpu.make_async_copy(ei_hbm.at[0, wid, 0], tarb[slot],
                              semi[slot]).wait()
        pltpu.make_async_copy(ei_hbm.at[0, wid, 0], srcb[slot],
                              semi[slot]).wait()

    def issue_gather(slot):
        pltpu.async_copy(feat_hbm.at[srcb[slot]], rows[slot], semg[slot])

    def wait_gather(slot):
        pltpu.make_async_copy(feat_hbm.at[srcb[slot]], rows[slot],
                              semg[slot]).wait()

    def wait_rows_scatter(slot):
        pltpu.make_async_copy(rows[slot], acc.at[sidx[slot]],
                              sema[slot]).wait()

    def wait_dacc_scatter(slot):
        pltpu.make_async_copy(dsrc[slot], dacc.at[sidx[slot]],
                              semd[slot]).wait()

    def compute_batch(slot, j):
        tb, sb, rw, dsv, six = (tarb[slot], srcb[slot], rows[slot],
                                dsrc[slot], sidx[slot])

        @pl.when(j >= 2)
        def _():
            wait_dacc_scatter(slot)
        for k in range(B // 16):
            six[pl.ds(k * 16, 16)] = tb[pl.ds(k * 16, 16)]
        for k in range(B // 16):
            it = tb[pl.ds(k * 16, 16)]
            isr = sb[pl.ds(k * 16, 16)]
            e = plsc.load_gather(ep0_v, [it]) + plsc.load_gather(ep1_v, [isr])
            e = jnp.where(e >= 0, e, NEG_SLOPE * e)
            wv = jnp.exp(e)
            plsc.store_scatter(dsv, [iota16 + k * 16, zeros16i], wv)

            @plsc.parallel_loop(0, 16, step=1, unroll=4)
            def _mul(r2):
                spl = wv.at[jnp.broadcast_to(r2, (16,))].get(
                    mode="promise_in_bounds")
                row = k * 16 + r2
                for c in range(C // 16):
                    rw[row, pl.ds(c * 16, 16)] = (
                        rw[row, pl.ds(c * 16, 16)] * spl)
        pltpu.async_copy(rw, acc.at[six], sema[slot], add=True)
        pltpu.async_copy(dsv, dacc.at[six], semd[slot], add=True)

    issue_idx(0, 0)
    issue_idx(1, 1)
    wait_idx(0)
    issue_gather(0)

    plsc.subcore_barrier()

    def _gbody(g, _):
        for b2 in range(2):
            j = g * 2 + b2
            wait_gather(b2)
            wait_idx(1 - b2)

            @pl.when(j >= 1)
            def _():
                wait_rows_scatter(1 - b2)
            issue_gather(1 - b2)
            compute_batch(b2, j)

            @pl.when(j + 2 < NB)
            def _():
                issue_idx(j + 2, b2)
        return 0
    lax.fori_loop(0, NB // 2, _gbody, 0)

    wait_gather(0)
    compute_batch(0, jnp.int32(NB - 1))

    wait_rows_scatter(0)
    wait_rows_scatter(1)
    wait_dacc_scatter(0)
    wait_dacc_scatter(1)

    plsc.subcore_barrier()
    pltpu.sync_copy(acc.at[pl.ds(base, RPT)], p_hbm.at[cid, pl.ds(base, RPT)])
    pltpu.sync_copy(dacc.at[pl.ds(base, RPT)],
                    dq_hbm.at[cid, pl.ds(base, RPT)])


_sc_edge_pass = functools.partial(
    pl.kernel,
    out_type=[
        jax.ShapeDtypeStruct((NC, N, C), jnp.float32),
        jax.ShapeDtypeStruct((NC, N, DW), jnp.float32),
    ],
    mesh=plsc.VectorSubcoreMesh(core_axis_name="c", subcore_axis_name="s",
                                num_cores=NC, num_subcores=NS),
    compiler_params=pltpu.CompilerParams(needs_layout_passes=False,
                                         use_tc_tiling_on_sc=False),
    scratch_types=[
        pltpu.VMEM((N,), jnp.float32),
        pltpu.VMEM((N,), jnp.float32),
        pltpu.VMEM((B,), jnp.int32),
        pltpu.VMEM((B,), jnp.int32),
        pltpu.VMEM((B,), jnp.int32),
        pltpu.VMEM((B,), jnp.int32),
        pltpu.VMEM((B, C), jnp.float32),
        pltpu.VMEM((B, C), jnp.float32),
        pltpu.VMEM((B, DW), jnp.float32),
        pltpu.VMEM((B, DW), jnp.float32),
        pltpu.VMEM((B,), jnp.int32),
        pltpu.VMEM((B,), jnp.int32),
        pltpu.VMEM_SHARED((N, C), jnp.float32),
        pltpu.VMEM_SHARED((N, DW), jnp.float32),
        pltpu.SemaphoreType.DMA,
        pltpu.SemaphoreType.DMA,
        pltpu.SemaphoreType.DMA,
        pltpu.SemaphoreType.DMA,
        pltpu.SemaphoreType.DMA,
        pltpu.SemaphoreType.DMA,
        pltpu.SemaphoreType.DMA,
        pltpu.SemaphoreType.DMA,
    ],
)(_sc_body)


def kernel(x, edge_index, W, b, att):
    ei4 = edge_index.astype(jnp.int32).reshape(2, NW, NB, B)
    att_pad = jnp.zeros((C, C), jnp.float32).at[:, 0:2].set(att)
    attT = att.T
    b2 = b.reshape(1, C)

    R = 1000
    feat, ep, ept = pl.pallas_call(
        _k1_body,
        grid=(N // R,),
        in_specs=[
            pl.BlockSpec((R, C), lambda i: (i, 0)),
            pl.BlockSpec((C, C), lambda i: (0, 0)),
            pl.BlockSpec((1, C), lambda i: (0, 0)),
            pl.BlockSpec((C, C), lambda i: (0, 0)),
            pl.BlockSpec((2, C), lambda i: (0, 0)),
        ],
        out_specs=[
            pl.BlockSpec((R, C), lambda i: (i, 0)),
            pl.BlockSpec((R, C), lambda i: (i, 0)),
            pl.BlockSpec((1, 2, R), lambda i: (i, 0, 0)),
        ],
        out_shape=[
            jax.ShapeDtypeStruct((N, C), jnp.float32),
            jax.ShapeDtypeStruct((N, C), jnp.float32),
            jax.ShapeDtypeStruct((N // R, 2, R), jnp.float32),
        ],
    )(x, W, b2, att_pad, attT)

    p, dq = _sc_edge_pass(ei4, ept, feat)

    out = pl.pallas_call(
        _k3_body,
        grid=(N // R,),
        in_specs=[
            pl.BlockSpec((NC, R, C), lambda i: (0, i, 0)),
            pl.BlockSpec((NC, R, DW), lambda i: (0, i, 0)),
            pl.BlockSpec((R, C), lambda i: (i, 0)),
            pl.BlockSpec((R, C), lambda i: (i, 0)),
        ],
        out_specs=pl.BlockSpec((R, C), lambda i: (i, 0)),
        out_shape=jax.ShapeDtypeStruct((N, C), jnp.float32),
    )(p, dq, feat, ep)
    return out


# speedup vs baseline: 1.0299x; 1.0299x over previous
import functools

import jax
import jax.numpy as jnp
from jax import lax
from jax.experimental import pallas as pl
from jax.experimental.pallas import tpu as pltpu
from jax.experimental.pallas import tpu_sc as plsc

N = 10000
E = 320000
C = 128
NEG_SLOPE = 0.2

NC = 2
NS = 16
NW = NC * NS
EPW = E // NW
B = 80
NB = EPW // B
RPT = N // NS
DW = 8

_HIGH = jax.lax.Precision.HIGHEST


def _k1_body(x_ref, w_ref, b_ref, ap_ref, at_ref, feat_ref, ep_ref, ept_ref):
    i = pl.program_id(0)
    xb = x_ref[...]
    feat = lax.dot_general(xb, w_ref[...], (((1,), (1,)), ((), ()))) + b_ref[...]
    feat_ref[...] = feat
    ep_ref[...] = lax.dot_general(feat, ap_ref[...], (((1,), (0,)), ((), ())))
    ept = lax.dot_general(at_ref[...], feat, (((1,), (1,)), ((), ())))
    ept_ref[0] = ept


def _k3_body(p_ref, dq_ref, feat_ref, ep_ref, out_ref):
    z = ep_ref[:, 0:1] + ep_ref[:, 1:2]
    es = jnp.exp(jnp.where(z >= 0, z, NEG_SLOPE * z))
    den = dq_ref[0, :, 0:1] + dq_ref[1, :, 0:1] + es
    num = p_ref[0] + p_ref[1] + es * feat_ref[...]
    out_ref[...] = num / den


def _sc_body(ei_hbm, ept_hbm, feat_hbm,
             p_hbm, dq_hbm,
             ep0_v, ep1_v, tarb0, tarb1, srcb0, srcb1, rows0, rows1,
             dsrc0, dsrc1, sidx0, sidx1,
             acc, dacc, semg0, semg1, semi0, semi1,
             sema0, sema1, semd0, semd1):
    cid = lax.axis_index("c")
    sid = lax.axis_index("s")
    wid = sid * NC + cid

    zeros16f = jnp.zeros((16,), jnp.float32)
    zeros16i = jnp.zeros((16,), jnp.int32)
    iota16 = lax.iota(jnp.int32, 16)

    tarb = (tarb0, tarb1)
    srcb = (srcb0, srcb1)
    rows = (rows0, rows1)
    dsrc = (dsrc0, dsrc1)
    sidx = (sidx0, sidx1)
    semg = (semg0, semg1)
    semi = (semi0, semi1)
    sema = (sema0, sema1)
    semd = (semd0, semd1)

    def _zr(i, _):
        for k in range(C // 16):
            rows0[i, pl.ds(k * 16, 16)] = zeros16f
        return 0
    lax.fori_loop(0, B, _zr, 0)

    def _zs(m, _):
        rowz = 2 * m + jnp.right_shift(iota16, 3)
        colz = jnp.bitwise_and(iota16, 7)
        plsc.store_scatter(dsrc0, [rowz, colz], zeros16f)
        plsc.store_scatter(dsrc1, [rowz, colz], zeros16f)
        return 0
    lax.fori_loop(0, B * DW // 16, _zs, 0)

    base = sid * RPT
    for i in range(RPT // B):
        pltpu.sync_copy(rows0, acc.at[pl.ds(base + i * B, B)])
        pltpu.sync_copy(dsrc0, dacc.at[pl.ds(base + i * B, B)])
    rem = RPT % B
    if rem:
        off = base + (RPT // B) * B
        pltpu.sync_copy(rows0.at[pl.ds(0, rem)], acc.at[pl.ds(off, rem)])
        pltpu.sync_copy(dsrc0.at[pl.ds(0, rem)], dacc.at[pl.ds(off, rem)])

    NCH = N // 1000
    for i2 in range(NCH):
        pltpu.async_copy(ept_hbm.at[i2, 0], ep0_v.at[pl.ds(i2 * 1000, 1000)],
                         semg0)
        pltpu.async_copy(ept_hbm.at[i2, 1], ep1_v.at[pl.ds(i2 * 1000, 1000)],
                         semg0)
    for i2 in range(NCH):
        pltpu.make_async_copy(ept_hbm.at[i2, 0],
                              ep0_v.at[pl.ds(i2 * 1000, 1000)], semg0).wait()
        pltpu.make_async_copy(ept_hbm.at[i2, 1],
                              ep1_v.at[pl.ds(i2 * 1000, 1000)], semg0).wait()

    def issue_idx(j, slot):
        pltpu.async_copy(ei_hbm.at[0, wid, j], tarb[slot], semi[slot])
        pltpu.async_copy(ei_hbm.at[1, wid, j], srcb[slot], semi[slot])

    def wait_idx(slot):
        pltpu.make_async_copy(ei_hbm.at[0, wid, 0], tarb[slot],
                              semi[slot]).wait()
        pltpu.make_async_copy(ei_hbm.at[0, wid, 0], srcb[slot],
                              semi[slot]).wait()

    def issue_gather(slot):
        pltpu.async_copy(feat_hbm.at[srcb[slot]], rows[slot], semg[slot])

    def wait_gather(slot):
        pltpu.make_async_copy(feat_hbm.at[srcb[slot]], rows[slot],
                              semg[slot]).wait()

    def wait_rows_scatter(slot):
        pltpu.make_async_copy(rows[slot], acc.at[sidx[slot]],
                              sema[slot]).wait()

    def wait_dacc_scatter(slot):
        pltpu.make_async_copy(dsrc[slot], dacc.at[sidx[slot]],
                              semd[slot]).wait()

    def compute_batch(slot, j):
        tb, sb, rw, dsv, six = (tarb[slot], srcb[slot], rows[slot],
                                dsrc[slot], sidx[slot])

        @pl.when(j >= 2)
        def _():
            wait_dacc_scatter(slot)
        for k in range(B // 16):
            six[pl.ds(k * 16, 16)] = tb[pl.ds(k * 16, 16)]
        for k in range(B // 16):
            it = tb[pl.ds(k * 16, 16)]
            isr = sb[pl.ds(k * 16, 16)]
            e = plsc.load_gather(ep0_v, [it]) + plsc.load_gather(ep1_v, [isr])
            e = jnp.where(e >= 0, e, NEG_SLOPE * e)
            wv = jnp.exp(e)
            plsc.store_scatter(dsv, [iota16 + k * 16, zeros16i], wv)

            @plsc.parallel_loop(0, 16, step=1, unroll=4)
            def _mul(r2):
                spl = wv.at[jnp.broadcast_to(r2, (16,))].get(
                    mode="promise_in_bounds")
                row = k * 16 + r2
                for c in range(C // 16):
                    rw[row, pl.ds(c * 16, 16)] = (
                        rw[row, pl.ds(c * 16, 16)] * spl)
        pltpu.async_copy(rw, acc.at[six], sema[slot], add=True)
        pltpu.async_copy(dsv, dacc.at[six], semd[slot], add=True)

    issue_idx(0, 0)
    issue_idx(1, 1)
    wait_idx(0)
    issue_gather(0)

    plsc.subcore_barrier()

    def _gbody(g, _):
        for b2 in range(2):
            j = g * 2 + b2
            wait_idx(1 - b2)

            @pl.when(j >= 1)
            def _():
                wait_rows_scatter(1 - b2)
            issue_gather(1 - b2)
            wait_gather(b2)
            compute_batch(b2, j)

            @pl.when(j + 2 < NB)
            def _():
                issue_idx(j + 2, b2)
        return 0
    lax.fori_loop(0, NB // 2, _gbody, 0)

    wait_gather(0)
    compute_batch(0, jnp.int32(NB - 1))

    wait_rows_scatter(0)
    wait_rows_scatter(1)
    wait_dacc_scatter(0)
    wait_dacc_scatter(1)

    plsc.subcore_barrier()
    pltpu.sync_copy(acc.at[pl.ds(base, RPT)], p_hbm.at[cid, pl.ds(base, RPT)])
    pltpu.sync_copy(dacc.at[pl.ds(base, RPT)],
                    dq_hbm.at[cid, pl.ds(base, RPT)])


_sc_edge_pass = functools.partial(
    pl.kernel,
    out_type=[
        jax.ShapeDtypeStruct((NC, N, C), jnp.float32),
        jax.ShapeDtypeStruct((NC, N, DW), jnp.float32),
    ],
    mesh=plsc.VectorSubcoreMesh(core_axis_name="c", subcore_axis_name="s",
                                num_cores=NC, num_subcores=NS),
    compiler_params=pltpu.CompilerParams(needs_layout_passes=False,
                                         use_tc_tiling_on_sc=False),
    scratch_types=[
        pltpu.VMEM((N,), jnp.float32),
        pltpu.VMEM((N,), jnp.float32),
        pltpu.VMEM((B,), jnp.int32),
        pltpu.VMEM((B,), jnp.int32),
        pltpu.VMEM((B,), jnp.int32),
        pltpu.VMEM((B,), jnp.int32),
        pltpu.VMEM((B, C), jnp.float32),
        pltpu.VMEM((B, C), jnp.float32),
        pltpu.VMEM((B, DW), jnp.float32),
        pltpu.VMEM((B, DW), jnp.float32),
        pltpu.VMEM((B,), jnp.int32),
        pltpu.VMEM((B,), jnp.int32),
        pltpu.VMEM_SHARED((N, C), jnp.float32),
        pltpu.VMEM_SHARED((N, DW), jnp.float32),
        pltpu.SemaphoreType.DMA,
        pltpu.SemaphoreType.DMA,
        pltpu.SemaphoreType.DMA,
        pltpu.SemaphoreType.DMA,
        pltpu.SemaphoreType.DMA,
        pltpu.SemaphoreType.DMA,
        pltpu.SemaphoreType.DMA,
        pltpu.SemaphoreType.DMA,
    ],
)(_sc_body)


def kernel(x, edge_index, W, b, att):
    ei4 = edge_index.astype(jnp.int32).reshape(2, NW, NB, B)
    att_pad = jnp.zeros((C, C), jnp.float32).at[:, 0:2].set(att)
    attT = att.T
    b2 = b.reshape(1, C)

    R = 1000
    feat, ep, ept = pl.pallas_call(
        _k1_body,
        grid=(N // R,),
        in_specs=[
            pl.BlockSpec((R, C), lambda i: (i, 0)),
            pl.BlockSpec((C, C), lambda i: (0, 0)),
            pl.BlockSpec((1, C), lambda i: (0, 0)),
            pl.BlockSpec((C, C), lambda i: (0, 0)),
            pl.BlockSpec((2, C), lambda i: (0, 0)),
        ],
        out_specs=[
            pl.BlockSpec((R, C), lambda i: (i, 0)),
            pl.BlockSpec((R, C), lambda i: (i, 0)),
            pl.BlockSpec((1, 2, R), lambda i: (i, 0, 0)),
        ],
        out_shape=[
            jax.ShapeDtypeStruct((N, C), jnp.float32),
            jax.ShapeDtypeStruct((N, C), jnp.float32),
            jax.ShapeDtypeStruct((N // R, 2, R), jnp.float32),
        ],
    )(x, W, b2, att_pad, attT)

    p, dq = _sc_edge_pass(ei4, ept, feat)

    out = pl.pallas_call(
        _k3_body,
        grid=(N // R,),
        in_specs=[
            pl.BlockSpec((NC, R, C), lambda i: (0, i, 0)),
            pl.BlockSpec((NC, R, DW), lambda i: (0, i, 0)),
            pl.BlockSpec((R, C), lambda i: (i, 0)),
            pl.BlockSpec((R, C), lambda i: (i, 0)),
        ],
        out_specs=pl.BlockSpec((R, C), lambda i: (i, 0)),
        out_shape=jax.ShapeDtypeStruct((N, C), jnp.float32),
    )(p, dq, feat, ep)
    return out
